# baseline (speedup 1.0000x reference)
import functools

import jax
import jax.numpy as jnp
from jax import lax
from jax.experimental import pallas as pl
from jax.experimental.pallas import tpu as pltpu
from jax.experimental.pallas import tpu_sc as plsc

N = 10000
E = 320000
F_IN = 128
F_HID = 128
F_OUT = 64


NC = 2
NS = 16
NW = NC * NS
C = 128
E_PAD = -(-E // (NW * C)) * NW * C
PT = E_PAD // (NW * C)
NPAD = N + 8
NBUF = 3
DIST = NBUF - 1
UNROLL = 12
R_MAIN = 632
R_LAST = N - (NS - 1) * R_MAIN


@functools.lru_cache(maxsize=None)
def _make_sc_agg(F, compute_deg):
  mesh = plsc.VectorSubcoreMesh(core_axis_name="c", subcore_axis_name="s",
                                num_cores=NC, num_subcores=NS)
  out_type = [jax.ShapeDtypeStruct((N, F), jnp.float32),
              jax.ShapeDtypeStruct((N, F), jnp.float32)]
  nib = 4 if compute_deg else 6
  scratch = (
      [pltpu.VMEM((2, C), jnp.int32) for _ in range(nib)]
      + [pltpu.VMEM((C, F), jnp.float32) for _ in range(NBUF)]
      + [pltpu.VMEM_SHARED((NPAD, F), jnp.float32)]
      + [pltpu.SemaphoreType.DMA for _ in range(nib + 2 * NBUF + 1)]
  )
  if compute_deg:
    out_type += [jax.ShapeDtypeStruct((NPAD,), jnp.float32),
                 jax.ShapeDtypeStruct((NPAD,), jnp.float32)]
    scratch += [
        pltpu.VMEM((C,), jnp.float32),
        pltpu.VMEM_SHARED((NPAD,), jnp.float32),
    ]

  def body(y_hbm, ei_hbm, et_hbm, z2d_hbm, z1_hbm, agg0_hbm, agg1_hbm, *rest):
    if compute_deg:
      deg0_hbm, deg1_hbm = rest[0], rest[1]
      rest = rest[2:]
    eiv = rest[:nib]
    rows = rest[nib:nib + NBUF]
    agg_sh = rest[nib + NBUF]
    sems = rest[nib + NBUF + 1:nib + NBUF + 1 + nib + 2 * NBUF + 1]
    isem = sems[:nib]
    gsem = sems[nib:nib + NBUF]
    ssem = sems[nib + NBUF:nib + 2 * NBUF]
    zsem = sems[nib + 2 * NBUF]
    if compute_deg:
      ones_v, deg_sh = rest[-2], rest[-1]
    c = lax.axis_index("c")
    s = lax.axis_index("s")
    wid = s * NC + c

    r0 = pl.multiple_of(s * R_MAIN, 8)

    def copy_rows(src_ref, dst_ref, sem=None):
      @pl.when(s < NS - 1)
      def _():
        sl = (pl.ds(r0, R_MAIN),)
        if sem is None:
          pltpu.sync_copy(src_ref.at[sl], dst_ref.at[sl])
        else:
          pltpu.async_copy(src_ref.at[sl], dst_ref.at[sl], sem)
      @pl.when(s == NS - 1)
      def _():
        sl = (pl.ds(r0, R_LAST),)
        if sem is None:
          pltpu.sync_copy(src_ref.at[sl], dst_ref.at[sl])
        else:
          pltpu.async_copy(src_ref.at[sl], dst_ref.at[sl], sem)

    def wait_rows(src_ref, dst_ref, sem):
      @pl.when(s < NS - 1)
      def _():
        sl = (pl.ds(r0, R_MAIN),)
        pltpu.make_async_copy(src_ref.at[sl], dst_ref.at[sl], sem).wait()
      @pl.when(s == NS - 1)
      def _():
        sl = (pl.ds(r0, R_LAST),)
        pltpu.make_async_copy(src_ref.at[sl], dst_ref.at[sl], sem).wait()

    def idx_src(j, tail):
      if tail:
        return et_hbm.at[:, pl.ds(pl.multiple_of(wid * C, C), C)]
      off = pl.multiple_of((j * NW + wid) * C, C)
      return ei_hbm.at[:, pl.ds(off, C)]

    def idx_load(j, ib, tail=False):
      pltpu.async_copy(idx_src(j, tail), eiv[ib], isem[ib])

    def idx_wait(j, ib, tail=False):
      pltpu.make_async_copy(idx_src(j, tail), eiv[ib], isem[ib]).wait()

    def drain_scatter(b, ib):
      pltpu.make_async_copy(rows[b], agg_sh.at[eiv[ib].at[1]], ssem[b]).wait()
      if compute_deg:
        pltpu.make_async_copy(ones_v, deg_sh.at[eiv[ib].at[1]], ssem[b]).wait()

    copy_rows(z2d_hbm, agg_sh, zsem)
    for k in range(DIST + 1):
      idx_load(k, k)
    for k in range(DIST):
      idx_wait(k, k)
      pltpu.async_copy(y_hbm.at[eiv[k].at[0]], rows[k], gsem[k])
    wait_rows(z2d_hbm, agg_sh, zsem)
    if compute_deg:
      @pl.when(s == 0)
      def _():
        pltpu.sync_copy(z1_hbm, deg_sh)
      ones16 = jnp.full((16,), 1.0, jnp.float32)
      for j in range(C // 16):
        ones_v[pl.ds(j * 16, 16)] = ones16
    plsc.subcore_barrier()

    def iter_body(j, k, tail_load=False, tail_wait=False):
      b = k % NBUF
      ib = k % nib
      pltpu.make_async_copy(y_hbm.at[eiv[ib].at[0]], rows[b], gsem[b]).wait()
      pltpu.async_copy(rows[b], agg_sh.at[eiv[ib].at[1]], ssem[b], add=True)
      if compute_deg:
        pltpu.async_copy(ones_v, deg_sh.at[eiv[ib].at[1]], ssem[b], add=True)
      @pl.when(j + DIST < PT)
      def _():
        @pl.when(j >= 1)
        def _():
          drain_scatter((k + NBUF - 1) % NBUF, (k + nib - 1) % nib)
        @pl.when(j + DIST + 1 < PT)
        def _():
          idx_load(j + DIST + 1, (k + DIST + 1) % nib, tail_load)
        idx_wait(j + DIST, (k + DIST) % nib, tail_wait)
        pltpu.async_copy(y_hbm.at[eiv[(k + DIST) % nib].at[0]],
                         rows[(k + DIST) % NBUF], gsem[(k + DIST) % NBUF])

    def block(q, carry):
      for k in range(UNROLL):
        iter_body(q * UNROLL + k, k)
      return carry
    lax.fori_loop(0, PT // UNROLL, block, 0)
    for j in range(UNROLL * (PT // UNROLL), PT):
      iter_body(j, j % UNROLL, tail_load=(j + DIST + 1 == PT - 1),
                tail_wait=(j + DIST == PT - 1))
    for j in range(max(0, PT - NBUF), PT):
      drain_scatter(j % NBUF, j % nib)

    plsc.subcore_barrier()

    @pl.when(c == 0)
    def _():
      copy_rows(agg_sh, agg0_hbm)
    @pl.when(c == 1)
    def _():
      copy_rows(agg_sh, agg1_hbm)
    if compute_deg:
      @pl.when((c == 0) & (s == 0))
      def _():
        pltpu.sync_copy(deg_sh, deg0_hbm)
      @pl.when((c == 1) & (s == 0))
      def _():
        pltpu.sync_copy(deg_sh, deg1_hbm)

  return pl.kernel(body, out_type=out_type, mesh=mesh, scratch_types=scratch)


BM = 1024
GRID = pl.cdiv(N, BM)


def _tc_self_body(h_ref, ws_ref, b_ref, o_ref):
  o_ref[...] = (jnp.dot(h_ref[...], ws_ref[...],
                        preferred_element_type=jnp.float32) + b_ref[...])


def _tc_self(h, ws, b):
  fi, fo = ws.shape
  return pl.pallas_call(
      _tc_self_body,
      grid=(GRID,),
      in_specs=[
          pl.BlockSpec((BM, fi), lambda i: (i, 0)),
          pl.BlockSpec((fi, fo), lambda i: (0, 0)),
          pl.BlockSpec((1, fo), lambda i: (0, 0)),
      ],
      out_specs=pl.BlockSpec((BM, fo), lambda i: (i, 0)),
      out_shape=jax.ShapeDtypeStruct((N, fo), jnp.float32),
  )(h, ws, b)


def _tc_combine_body(relu, s_ref, a0_ref, a1_ref, d0_ref, d1_ref, wn_ref,
                     o_ref):
  deg = jnp.maximum(d0_ref[...] + d1_ref[...], 1.0)
  h_neigh = (a0_ref[...] + a1_ref[...]) / deg[:, None]
  o = s_ref[...] + jnp.dot(h_neigh, wn_ref[...],
                           preferred_element_type=jnp.float32)
  o_ref[...] = jax.nn.relu(o) if relu else o


def _tc_combine(s, a0, a1, d0, d1, wn, relu):
  fi, fo = wn.shape
  return pl.pallas_call(
      functools.partial(_tc_combine_body, relu),
      grid=(GRID,),
      in_specs=[
          pl.BlockSpec((BM, fo), lambda i: (i, 0)),
          pl.BlockSpec((BM, fi), lambda i: (i, 0)),
          pl.BlockSpec((BM, fi), lambda i: (i, 0)),
          pl.BlockSpec((BM,), lambda i: (i,)),
          pl.BlockSpec((BM,), lambda i: (i,)),
          pl.BlockSpec((fi, fo), lambda i: (0, 0)),
      ],
      out_specs=pl.BlockSpec((BM, fo), lambda i: (i, 0)),
      out_shape=jax.ShapeDtypeStruct((N, fo), jnp.float32),
  )(s, a0, a1, d0, d1, wn)


def kernel(x, edge_index, W_self0, W_neigh0, b0, W_self1, W_neigh1, b1,
           W_self2, W_neigh2, b2):
  pad = E_PAD - E
  tail_off = (PT - 1) * NW * C
  ar = jnp.arange(pad, dtype=jnp.int32)
  pad_pair = jnp.stack([(ar * 97) % N, N + (ar % 8)])
  ei_tail = jnp.concatenate([edge_index[:, tail_off:], pad_pair], axis=1)
  z2d128 = jnp.zeros((N, F_HID), jnp.float32)
  z1 = jnp.zeros((NPAD,), jnp.float32)

  s1 = _tc_self(x, W_self0, b0.reshape(1, -1))
  a0, a1, d0, d1 = _make_sc_agg(F_IN, True)(x, edge_index, ei_tail, z2d128, z1)
  d0 = d0[:N]
  d1 = d1[:N]
  h1 = _tc_combine(s1, a0, a1, d0, d1, W_neigh0, relu=True)
  s2 = _tc_self(h1, W_self1, b1.reshape(1, -1))
  a0b, a1b = _make_sc_agg(F_HID, False)(h1, edge_index, ei_tail, z2d128, z1)
  h2 = _tc_combine(s2, a0b, a1b, d0, d1, W_neigh1, relu=True)
  s3 = _tc_self(h2, W_self2, b2.reshape(1, -1))
  a0c, a1c = _make_sc_agg(F_HID, False)(h2, edge_index, ei_tail, z2d128, z1)
  return _tc_combine(s3, a0c, a1c, d0, d1, W_neigh2, relu=False)
